# baseline (speedup 1.0000x reference)
import functools

import jax
import jax.numpy as jnp
from jax import lax
from jax.experimental import pallas as pl
from jax.experimental.pallas import tpu as pltpu
from jax.experimental.pallas import tpu_sc as plsc

N_NODES = 10000
DIM = 128
N_EDGES = 320000

NUM_CORES = 2
NUM_SUBCORES = 16
NUM_WORKERS = NUM_CORES * NUM_SUBCORES

DCHUNK = 128
DEG_CHUNKS = 80
DEG_EDGES = NUM_WORKERS * DEG_CHUNKS * DCHUNK
CHUNK = 128
F0_CHUNKS = 160
F1_CHUNKS = 0
F_MAX = 160
PHASE_CHUNKS = 40
PHASES = F_MAX // PHASE_CHUNKS

TROWS = 10240
ZROWS = TROWS // NUM_SUBCORES
ZBUF = 64
NBUF = 2
DEG_W = 16

_MESH = plsc.VectorSubcoreMesh(core_axis_name="c", subcore_axis_name="s")


def _sc_degree(dst_w):
    @functools.partial(
        pl.kernel,
        out_type=jax.ShapeDtypeStruct((NUM_CORES, TROWS, DEG_W), jnp.float32),
        mesh=_MESH,
        scratch_types=[
            pltpu.VMEM((DEG_CHUNKS, DCHUNK), jnp.int32),
            pltpu.VMEM((DCHUNK, DEG_W), jnp.float32),
            pltpu.VMEM((ZBUF, DEG_W), jnp.float32),
            pltpu.VMEM_SHARED((TROWS, DEG_W), jnp.float32),
        ],
    )
    def deg_kernel(dst_hbm, out_hbm, dst_v, ones_v, zero_v, acc_sh):
        c = lax.axis_index("c")
        s = lax.axis_index("s")
        wid = s * NUM_CORES + c

        @pl.loop(0, DCHUNK)
        def _(i):
            ones_v[i, :] = jnp.full((DEG_W,), 1.0, jnp.float32)

        @pl.loop(0, ZBUF)
        def _(i):
            zero_v[i, :] = jnp.zeros((DEG_W,), jnp.float32)

        @pl.loop(0, ZROWS // ZBUF)
        def _(i):
            pltpu.sync_copy(zero_v, acc_sh.at[pl.ds(s * ZROWS + i * ZBUF, ZBUF)])

        pltpu.sync_copy(dst_hbm.at[wid], dst_v)
        plsc.subcore_barrier()

        @pl.loop(0, DEG_CHUNKS)
        def _(j):
            pltpu.sync_copy(ones_v, acc_sh.at[dst_v.at[j]], add=True)

        plsc.subcore_barrier()
        pltpu.sync_copy(acc_sh.at[pl.ds(s * ZROWS, ZROWS)],
                        out_hbm.at[c].at[pl.ds(s * ZROWS, ZROWS)])

    return deg_kernel(dst_w)


def _sc_scatter(tab, src_w, dst_w):
    @functools.partial(
        pl.kernel,
        out_type=jax.ShapeDtypeStruct((NUM_CORES, TROWS, DIM), jnp.float32),
        mesh=_MESH,
        scratch_types=[
            pltpu.VMEM((PHASE_CHUNKS, CHUNK), jnp.int32),
            pltpu.VMEM((PHASE_CHUNKS, CHUNK), jnp.int32),
            pltpu.VMEM((NBUF, CHUNK, DIM), jnp.float32),
            pltpu.VMEM_SHARED((TROWS, DIM), jnp.float32),
            [pltpu.SemaphoreType.DMA] * NBUF,
            [pltpu.SemaphoreType.DMA] * NBUF,
        ],
    )
    def scat_kernel(tab_hbm, src_hbm, dst_hbm, out_hbm,
                    src_v, dst_v, rows_v, acc_sh, semg, sems):
        c = lax.axis_index("c")
        s = lax.axis_index("s")
        wid = s * NUM_CORES + c
        n_c = jnp.where(c == 0, F0_CHUNKS, F1_CHUNKS)

        @pl.loop(0, ZBUF)
        def _(i):
            @pl.loop(0, DIM // 16)
            def _(k):
                rows_v[0, i, pl.ds(k * 16, 16)] = jnp.zeros((16,), jnp.float32)

        @pl.loop(0, ZROWS // ZBUF)
        def _(i):
            pltpu.sync_copy(rows_v.at[0, pl.ds(0, ZBUF)],
                            acc_sh.at[pl.ds(s * ZROWS + i * ZBUF, ZBUF)])

        plsc.subcore_barrier()

        @pl.loop(0, PHASES)
        def _(p):
            base = p * PHASE_CHUNKS

            @pl.when(base < n_c)
            def _():
                pltpu.sync_copy(src_hbm.at[wid].at[pl.ds(base, PHASE_CHUNKS)],
                                src_v)
                pltpu.sync_copy(dst_hbm.at[wid].at[pl.ds(base, PHASE_CHUNKS)],
                                dst_v)

            for b in range(NBUF):
                @pl.when(base + b < n_c)
                def _():
                    pltpu.async_copy(tab_hbm.at[src_v.at[b]], rows_v.at[b],
                                     semg[b])

            @pl.loop(0, PHASE_CHUNKS, step=NBUF)
            def _(j):
                for b in range(NBUF):
                    @pl.when(base + j + b < n_c)
                    def _():
                        pltpu.make_async_copy(
                            tab_hbm.at[src_v.at[j + b]], rows_v.at[b],
                            semg[b]).wait()
                        pltpu.async_copy(
                            rows_v.at[b], acc_sh.at[dst_v.at[j + b]], sems[b],
                            add=True)
                for b in range(NBUF):
                    @pl.when(base + j + b < n_c)
                    def _():
                        pltpu.make_async_copy(
                            rows_v.at[b], acc_sh.at[dst_v.at[j + b]],
                            sems[b]).wait()

                    @pl.when(jnp.logical_and(j + NBUF + b < PHASE_CHUNKS,
                                             base + j + NBUF + b < n_c))
                    def _():
                        pltpu.async_copy(
                            tab_hbm.at[src_v.at[j + NBUF + b]], rows_v.at[b],
                            semg[b])

        plsc.subcore_barrier()
        pltpu.sync_copy(acc_sh.at[pl.ds(s * ZROWS, ZROWS)],
                        out_hbm.at[c].at[pl.ds(s * ZROWS, ZROWS)])

    return scat_kernel(tab, src_w, dst_w)


def _tc_prep1(x, W1, deg_parts):
    def body(x_ref, w_ref, dp_ref, tab_ref, dinv_ref):
        deg = dp_ref[0, :N_NODES, 0:1] + dp_ref[1, :N_NODES, 0:1] + 1.0
        dinv = lax.rsqrt(deg)
        xw = jnp.dot(x_ref[...], w_ref[...],
                     preferred_element_type=jnp.float32,
                     precision=lax.Precision.HIGHEST)
        tab_ref[:N_NODES, :] = xw * dinv
        tab_ref[N_NODES:, :] = jnp.zeros((TROWS - N_NODES, DIM), jnp.float32)
        dinv_ref[...] = jnp.broadcast_to(dinv, (N_NODES, DIM))

    return pl.pallas_call(
        body,
        out_shape=[jax.ShapeDtypeStruct((TROWS, DIM), jnp.float32),
                   jax.ShapeDtypeStruct((N_NODES, DIM), jnp.float32)],
    )(x, W1, deg_parts)


def _tc_mid(acc1, tab1, x, dinv_b, b1, gamma1, beta1, W2):
    def body(acc_ref, tab_ref, x_ref, dinv_ref, b_ref, gam_ref, bet_ref,
             w_ref, h_ref, tab2_ref):
        agg = (acc_ref[0, :N_NODES, :] + acc_ref[1, :N_NODES, :]
               + tab_ref[:N_NODES, :])
        g = dinv_ref[...] * agg + b_ref[...]
        mu = jnp.mean(g, axis=0, keepdims=True)
        var = jnp.mean((g - mu) ** 2, axis=0, keepdims=True)
        bn = (g - mu) * lax.rsqrt(var + 1e-5) * gam_ref[...] + bet_ref[...]
        h = x_ref[...] + jnp.maximum(bn, 0.0)
        h_ref[...] = h
        xw2 = jnp.dot(h, w_ref[...],
                      preferred_element_type=jnp.float32,
                      precision=lax.Precision.HIGHEST)
        tab2_ref[:N_NODES, :] = xw2 * dinv_ref[...]
        tab2_ref[N_NODES:, :] = jnp.zeros((TROWS - N_NODES, DIM), jnp.float32)

    return pl.pallas_call(
        body,
        out_shape=[jax.ShapeDtypeStruct((N_NODES, DIM), jnp.float32),
                   jax.ShapeDtypeStruct((TROWS, DIM), jnp.float32)],
    )(acc1, tab1, x, dinv_b, b1, gamma1, beta1, W2)


def _tc_post(acc2, tab2, h1, dinv_b, b2, gamma2, beta2):
    def body(acc_ref, tab_ref, h_ref, dinv_ref, b_ref, gam_ref, bet_ref,
             out_ref):
        agg = (acc_ref[0, :N_NODES, :] + acc_ref[1, :N_NODES, :]
               + tab_ref[:N_NODES, :])
        g = dinv_ref[...] * agg + b_ref[...]
        mu = jnp.mean(g, axis=0, keepdims=True)
        var = jnp.mean((g - mu) ** 2, axis=0, keepdims=True)
        bn = (g - mu) * lax.rsqrt(var + 1e-5) * gam_ref[...] + bet_ref[...]
        h = h_ref[...] + bn
        nrm = jnp.maximum(jnp.sqrt(jnp.sum(h * h, axis=1, keepdims=True)),
                          1e-12)
        out_ref[...] = h / nrm

    return pl.pallas_call(
        body,
        out_shape=jax.ShapeDtypeStruct((N_NODES, DIM), jnp.float32),
    )(acc2, tab2, h1, dinv_b, b2, gamma2, beta2)


def kernel(x, W1, b1, gamma1, beta1, W2, b2, gamma2, beta2, edge_index):
    ei = edge_index.astype(jnp.int32)
    pad = jnp.full((DEG_EDGES - N_EDGES,), N_NODES, jnp.int32)
    src_p = jnp.concatenate([ei[0], pad])
    dst_p = jnp.concatenate([ei[1], pad])

    def _split(e):
        e3 = e.reshape(NUM_SUBCORES, F0_CHUNKS + F1_CHUNKS, CHUNK)
        c0 = e3[:, :F0_CHUNKS]
        c1 = jnp.pad(e3[:, F0_CHUNKS:],
                     ((0, 0), (0, F_MAX - F1_CHUNKS), (0, 0)),
                     constant_values=N_NODES)
        return jnp.stack([c0, c1], axis=1).reshape(
            NUM_WORKERS, F_MAX, CHUNK)

    src_w = _split(src_p)
    dst_w = _split(dst_p)
    dst_d = dst_p.reshape(NUM_WORKERS, DEG_CHUNKS, DCHUNK)

    deg_parts = _sc_degree(dst_d)
    tab1, dinv_b = _tc_prep1(x, W1, deg_parts)
    acc1 = _sc_scatter(tab1, src_w, dst_w)
    h1, tab2 = _tc_mid(acc1, tab1, x, dinv_b,
                       b1.reshape(1, DIM), gamma1.reshape(1, DIM),
                       beta1.reshape(1, DIM), W2)
    acc2 = _sc_scatter(tab2, src_w, dst_w)
    return _tc_post(acc2, tab2, h1, dinv_b,
                    b2.reshape(1, DIM), gamma2.reshape(1, DIM),
                    beta2.reshape(1, DIM))


# speedup vs baseline: 1.4572x; 1.4572x over previous
import functools

import jax
import jax.numpy as jnp
from jax import lax
from jax.experimental import pallas as pl
from jax.experimental.pallas import tpu as pltpu
from jax.experimental.pallas import tpu_sc as plsc

N_NODES = 10000
DIM = 128
N_EDGES = 320000

NUM_CORES = 2
NUM_SUBCORES = 16
NUM_WORKERS = NUM_CORES * NUM_SUBCORES

DCHUNK = 128
DEG_CHUNKS = 80
DEG_EDGES = NUM_WORKERS * DEG_CHUNKS * DCHUNK
CHUNK = 128
F0_CHUNKS = 120
F1_CHUNKS = 40
F_MAX = F0_CHUNKS
PHASE_CHUNKS = 40
PHASES = F_MAX // PHASE_CHUNKS
PHASES_1 = F1_CHUNKS // PHASE_CHUNKS

TROWS = 10240
ZROWS = TROWS // NUM_SUBCORES
ZBUF = 64
NBUF = 2
DEG_W = 16

_MESH = plsc.VectorSubcoreMesh(core_axis_name="c", subcore_axis_name="s")


def _sc_degree(dst_w):
    @functools.partial(
        pl.kernel,
        out_type=jax.ShapeDtypeStruct((NUM_CORES, TROWS, DEG_W), jnp.float32),
        mesh=_MESH,
        scratch_types=[
            pltpu.VMEM((DEG_CHUNKS, DCHUNK), jnp.int32),
            pltpu.VMEM((DCHUNK, DEG_W), jnp.float32),
            pltpu.VMEM((ZBUF, DEG_W), jnp.float32),
            pltpu.VMEM_SHARED((TROWS, DEG_W), jnp.float32),
        ],
    )
    def deg_kernel(dst_hbm, out_hbm, dst_v, ones_v, zero_v, acc_sh):
        c = lax.axis_index("c")
        s = lax.axis_index("s")
        wid = s * NUM_CORES + c

        @pl.loop(0, DCHUNK)
        def _(i):
            ones_v[i, :] = jnp.full((DEG_W,), 1.0, jnp.float32)

        @pl.loop(0, ZBUF)
        def _(i):
            zero_v[i, :] = jnp.zeros((DEG_W,), jnp.float32)

        @pl.loop(0, ZROWS // ZBUF)
        def _(i):
            pltpu.sync_copy(zero_v, acc_sh.at[pl.ds(s * ZROWS + i * ZBUF, ZBUF)])

        pltpu.sync_copy(dst_hbm.at[wid], dst_v)
        plsc.subcore_barrier()

        @pl.loop(0, DEG_CHUNKS)
        def _(j):
            pltpu.sync_copy(ones_v, acc_sh.at[dst_v.at[j]], add=True)

        plsc.subcore_barrier()
        pltpu.sync_copy(acc_sh.at[pl.ds(s * ZROWS, ZROWS)],
                        out_hbm.at[c].at[pl.ds(s * ZROWS, ZROWS)])

    return deg_kernel(dst_w)


def _sc_scatter(tab, src_w, dst_w):
    @functools.partial(
        pl.kernel,
        out_type=jax.ShapeDtypeStruct((NUM_CORES, TROWS, DIM), jnp.float32),
        mesh=_MESH,
        scratch_types=[
            pltpu.VMEM((PHASE_CHUNKS, CHUNK), jnp.int32),
            pltpu.VMEM((PHASE_CHUNKS, CHUNK), jnp.int32),
            pltpu.VMEM((NBUF, CHUNK, DIM), jnp.float32),
            pltpu.VMEM_SHARED((TROWS, DIM), jnp.float32),
            [pltpu.SemaphoreType.DMA] * NBUF,
        ],
    )
    def scat_kernel(tab_hbm, src_hbm, dst_hbm, out_hbm,
                    src_v, dst_v, rows_v, acc_sh, semg):
        c = lax.axis_index("c")
        s = lax.axis_index("s")
        wid = s * NUM_CORES + c
        n_c = jnp.where(c == 0, F0_CHUNKS, F1_CHUNKS)

        @pl.loop(0, ZBUF)
        def _(i):
            @pl.loop(0, DIM // 16)
            def _(k):
                rows_v[0, i, pl.ds(k * 16, 16)] = jnp.zeros((16,), jnp.float32)

        @pl.loop(0, ZROWS // ZBUF)
        def _(i):
            pltpu.sync_copy(rows_v.at[0, pl.ds(0, ZBUF)],
                            acc_sh.at[pl.ds(s * ZROWS + i * ZBUF, ZBUF)])

        plsc.subcore_barrier()

        nph = jnp.where(c == 0, PHASES, PHASES_1)

        @pl.loop(0, PHASES)
        def _(p):
            @pl.when(p < nph)
            def _():
                base = p * PHASE_CHUNKS
                pltpu.sync_copy(src_hbm.at[wid].at[pl.ds(base, PHASE_CHUNKS)],
                                src_v)
                pltpu.sync_copy(dst_hbm.at[wid].at[pl.ds(base, PHASE_CHUNKS)],
                                dst_v)

                for b in range(NBUF):
                    pltpu.async_copy(tab_hbm.at[src_v.at[b]], rows_v.at[b],
                                     semg[b])

                @pl.loop(0, PHASE_CHUNKS, step=NBUF)
                def _(j):
                    for b in range(NBUF):
                        pltpu.make_async_copy(
                            tab_hbm.at[src_v.at[j + b]], rows_v.at[b],
                            semg[b]).wait()
                        pltpu.sync_copy(
                            rows_v.at[b], acc_sh.at[dst_v.at[j + b]],
                            add=True)

                        @pl.when(j + NBUF + b < PHASE_CHUNKS)
                        def _():
                            pltpu.async_copy(
                                tab_hbm.at[src_v.at[j + NBUF + b]],
                                rows_v.at[b], semg[b])

        plsc.subcore_barrier()
        pltpu.sync_copy(acc_sh.at[pl.ds(s * ZROWS, ZROWS)],
                        out_hbm.at[c].at[pl.ds(s * ZROWS, ZROWS)])

    return scat_kernel(tab, src_w, dst_w)


def _tc_prep1(x, W1, deg_parts):
    def body(x_ref, w_ref, dp_ref, tab_ref, dinv_ref):
        deg = dp_ref[0, :N_NODES, 0:1] + dp_ref[1, :N_NODES, 0:1] + 1.0
        dinv = lax.rsqrt(deg)
        xw = jnp.dot(x_ref[...], w_ref[...],
                     preferred_element_type=jnp.float32,
                     precision=lax.Precision.HIGHEST)
        tab_ref[:N_NODES, :] = xw * dinv
        tab_ref[N_NODES:, :] = jnp.zeros((TROWS - N_NODES, DIM), jnp.float32)
        dinv_ref[...] = jnp.broadcast_to(dinv, (N_NODES, DIM))

    return pl.pallas_call(
        body,
        out_shape=[jax.ShapeDtypeStruct((TROWS, DIM), jnp.float32),
                   jax.ShapeDtypeStruct((N_NODES, DIM), jnp.float32)],
    )(x, W1, deg_parts)


def _tc_mid(acc1, tab1, x, dinv_b, b1, gamma1, beta1, W2):
    def body(acc_ref, tab_ref, x_ref, dinv_ref, b_ref, gam_ref, bet_ref,
             w_ref, h_ref, tab2_ref):
        agg = (acc_ref[0, :N_NODES, :] + acc_ref[1, :N_NODES, :]
               + tab_ref[:N_NODES, :])
        g = dinv_ref[...] * agg + b_ref[...]
        mu = jnp.mean(g, axis=0, keepdims=True)
        var = jnp.mean((g - mu) ** 2, axis=0, keepdims=True)
        bn = (g - mu) * lax.rsqrt(var + 1e-5) * gam_ref[...] + bet_ref[...]
        h = x_ref[...] + jnp.maximum(bn, 0.0)
        h_ref[...] = h
        xw2 = jnp.dot(h, w_ref[...],
                      preferred_element_type=jnp.float32,
                      precision=lax.Precision.HIGHEST)
        tab2_ref[:N_NODES, :] = xw2 * dinv_ref[...]
        tab2_ref[N_NODES:, :] = jnp.zeros((TROWS - N_NODES, DIM), jnp.float32)

    return pl.pallas_call(
        body,
        out_shape=[jax.ShapeDtypeStruct((N_NODES, DIM), jnp.float32),
                   jax.ShapeDtypeStruct((TROWS, DIM), jnp.float32)],
    )(acc1, tab1, x, dinv_b, b1, gamma1, beta1, W2)


def _tc_post(acc2, tab2, h1, dinv_b, b2, gamma2, beta2):
    def body(acc_ref, tab_ref, h_ref, dinv_ref, b_ref, gam_ref, bet_ref,
             out_ref):
        agg = (acc_ref[0, :N_NODES, :] + acc_ref[1, :N_NODES, :]
               + tab_ref[:N_NODES, :])
        g = dinv_ref[...] * agg + b_ref[...]
        mu = jnp.mean(g, axis=0, keepdims=True)
        var = jnp.mean((g - mu) ** 2, axis=0, keepdims=True)
        bn = (g - mu) * lax.rsqrt(var + 1e-5) * gam_ref[...] + bet_ref[...]
        h = h_ref[...] + bn
        nrm = jnp.maximum(jnp.sqrt(jnp.sum(h * h, axis=1, keepdims=True)),
                          1e-12)
        out_ref[...] = h / nrm

    return pl.pallas_call(
        body,
        out_shape=jax.ShapeDtypeStruct((N_NODES, DIM), jnp.float32),
    )(acc2, tab2, h1, dinv_b, b2, gamma2, beta2)


def kernel(x, W1, b1, gamma1, beta1, W2, b2, gamma2, beta2, edge_index):
    ei = edge_index.astype(jnp.int32)
    pad = jnp.full((DEG_EDGES - N_EDGES,), N_NODES, jnp.int32)
    src_p = jnp.concatenate([ei[0], pad])
    dst_p = jnp.concatenate([ei[1], pad])

    def _split(e):
        e3 = e.reshape(NUM_SUBCORES, F0_CHUNKS + F1_CHUNKS, CHUNK)
        c0 = e3[:, :F0_CHUNKS]
        c1 = jnp.pad(e3[:, F0_CHUNKS:],
                     ((0, 0), (0, F_MAX - F1_CHUNKS), (0, 0)),
                     constant_values=N_NODES)
        return jnp.stack([c0, c1], axis=1).reshape(
            NUM_WORKERS, F_MAX, CHUNK)

    src_w = _split(src_p)
    dst_w = _split(dst_p)
    dst_d = dst_p.reshape(NUM_WORKERS, DEG_CHUNKS, DCHUNK)

    deg_parts = _sc_degree(dst_d)
    tab1, dinv_b = _tc_prep1(x, W1, deg_parts)
    acc1 = _sc_scatter(tab1, src_w, dst_w)
    h1, tab2 = _tc_mid(acc1, tab1, x, dinv_b,
                       b1.reshape(1, DIM), gamma1.reshape(1, DIM),
                       beta1.reshape(1, DIM), W2)
    acc2 = _sc_scatter(tab2, src_w, dst_w)
    return _tc_post(acc2, tab2, h1, dinv_b,
                    b2.reshape(1, DIM), gamma2.reshape(1, DIM),
                    beta2.reshape(1, DIM))


# speedup vs baseline: 1.4804x; 1.0160x over previous
import functools

import jax
import jax.numpy as jnp
from jax import lax
from jax.experimental import pallas as pl
from jax.experimental.pallas import tpu as pltpu
from jax.experimental.pallas import tpu_sc as plsc

N_NODES = 10000
DIM = 128
N_EDGES = 320000

NUM_CORES = 2
NUM_SUBCORES = 16
NUM_WORKERS = NUM_CORES * NUM_SUBCORES

DCHUNK = 128
DEG_CHUNKS = 80
DEG_EDGES = NUM_WORKERS * DEG_CHUNKS * DCHUNK
CHUNK = 128
F0_CHUNKS = 120
F1_CHUNKS = 40
F_MAX = F0_CHUNKS
PHASE_CHUNKS = 40
PHASES = F_MAX // PHASE_CHUNKS
PHASES_1 = F1_CHUNKS // PHASE_CHUNKS

TROWS = 10240
ZROWS = TROWS // NUM_SUBCORES
ZBUF = 64
NBUF = 2
DEG_W = 16

_MESH = plsc.VectorSubcoreMesh(core_axis_name="c", subcore_axis_name="s")


def _sc_degree(dst_w):
    @functools.partial(
        pl.kernel,
        out_type=jax.ShapeDtypeStruct((NUM_CORES, TROWS, DEG_W), jnp.float32),
        mesh=_MESH,
        scratch_types=[
            pltpu.VMEM((DEG_CHUNKS, DCHUNK), jnp.int32),
            pltpu.VMEM((DCHUNK, DEG_W), jnp.float32),
            pltpu.VMEM((ZBUF, DEG_W), jnp.float32),
            pltpu.VMEM_SHARED((TROWS, DEG_W), jnp.float32),
        ],
    )
    def deg_kernel(dst_hbm, out_hbm, dst_v, ones_v, zero_v, acc_sh):
        c = lax.axis_index("c")
        s = lax.axis_index("s")
        wid = s * NUM_CORES + c

        @pl.loop(0, DCHUNK)
        def _(i):
            ones_v[i, :] = jnp.full((DEG_W,), 1.0, jnp.float32)

        @pl.loop(0, ZBUF)
        def _(i):
            zero_v[i, :] = jnp.zeros((DEG_W,), jnp.float32)

        @pl.loop(0, ZROWS // ZBUF)
        def _(i):
            pltpu.sync_copy(zero_v, acc_sh.at[pl.ds(s * ZROWS + i * ZBUF, ZBUF)])

        pltpu.sync_copy(dst_hbm.at[wid], dst_v)
        plsc.subcore_barrier()

        @pl.loop(0, DEG_CHUNKS)
        def _(j):
            pltpu.sync_copy(ones_v, acc_sh.at[dst_v.at[j]], add=True)

        plsc.subcore_barrier()
        pltpu.sync_copy(acc_sh.at[pl.ds(s * ZROWS, ZROWS)],
                        out_hbm.at[c].at[pl.ds(s * ZROWS, ZROWS)])

    return deg_kernel(dst_w)


def _sc_scatter(tab, src_w, dst_w):
    @functools.partial(
        pl.kernel,
        out_type=jax.ShapeDtypeStruct((NUM_CORES, TROWS, DIM), jnp.float32),
        mesh=_MESH,
        scratch_types=[
            pltpu.VMEM((PHASE_CHUNKS, CHUNK), jnp.int32),
            pltpu.VMEM((PHASE_CHUNKS, CHUNK), jnp.int32),
            pltpu.VMEM((NBUF, CHUNK, DIM), jnp.float32),
            pltpu.VMEM_SHARED((TROWS, DIM), jnp.float32),
            [pltpu.SemaphoreType.DMA] * NBUF,
        ],
    )
    def scat_kernel(tab_hbm, src_hbm, dst_hbm, out_hbm,
                    src_v, dst_v, rows_v, acc_sh, semg):
        c = lax.axis_index("c")
        s = lax.axis_index("s")
        wid = s * NUM_CORES + c
        n_c = jnp.where(c == 0, F0_CHUNKS, F1_CHUNKS)

        @pl.loop(0, ZBUF)
        def _(i):
            @pl.loop(0, DIM // 16)
            def _(k):
                rows_v[0, i, pl.ds(k * 16, 16)] = jnp.zeros((16,), jnp.float32)

        @pl.loop(0, ZROWS // ZBUF)
        def _(i):
            pltpu.sync_copy(rows_v.at[0, pl.ds(0, ZBUF)],
                            acc_sh.at[pl.ds(s * ZROWS + i * ZBUF, ZBUF)])

        plsc.subcore_barrier()

        nph = jnp.where(c == 0, PHASES, PHASES_1)

        @pl.loop(0, PHASES)
        def _(p):
            @pl.when(p < nph)
            def _():
                base = p * PHASE_CHUNKS
                pltpu.sync_copy(src_hbm.at[wid].at[pl.ds(base, PHASE_CHUNKS)],
                                src_v)
                pltpu.sync_copy(dst_hbm.at[wid].at[pl.ds(base, PHASE_CHUNKS)],
                                dst_v)

                for b in range(NBUF):
                    pltpu.async_copy(tab_hbm.at[src_v.at[b]], rows_v.at[b],
                                     semg[b])

                @pl.loop(0, PHASE_CHUNKS, step=NBUF)
                def _(j):
                    for b in range(NBUF):
                        pltpu.make_async_copy(
                            tab_hbm.at[src_v.at[j + b]], rows_v.at[b],
                            semg[b]).wait()
                        pltpu.sync_copy(
                            rows_v.at[b], acc_sh.at[dst_v.at[j + b]],
                            add=True)

                        @pl.when(j + NBUF + b < PHASE_CHUNKS)
                        def _():
                            pltpu.async_copy(
                                tab_hbm.at[src_v.at[j + NBUF + b]],
                                rows_v.at[b], semg[b])

        plsc.subcore_barrier()
        pltpu.sync_copy(acc_sh.at[pl.ds(s * ZROWS, ZROWS)],
                        out_hbm.at[c].at[pl.ds(s * ZROWS, ZROWS)])

    return scat_kernel(tab, src_w, dst_w)


def _tc_mm1(x, W1):
    def body(x_ref, w_ref, xw_ref):
        xw_ref[...] = jnp.dot(x_ref[...], w_ref[...],
                              preferred_element_type=jnp.float32,
                              precision=lax.Precision.HIGHEST)

    return pl.pallas_call(
        body,
        out_shape=jax.ShapeDtypeStruct((N_NODES, DIM), jnp.float32),
    )(x, W1)


def _tc_prep1(xw1, deg_parts):
    def body(xw_ref, dp_ref, tab_ref, dinv_ref):
        deg = dp_ref[0, :N_NODES, 0:1] + dp_ref[1, :N_NODES, 0:1] + 1.0
        dinv = lax.rsqrt(deg)
        tab_ref[:N_NODES, :] = xw_ref[...] * dinv
        tab_ref[N_NODES:, :] = jnp.zeros((TROWS - N_NODES, DIM), jnp.float32)
        dinv_ref[...] = jnp.broadcast_to(dinv, (N_NODES, DIM))

    return pl.pallas_call(
        body,
        out_shape=[jax.ShapeDtypeStruct((TROWS, DIM), jnp.float32),
                   jax.ShapeDtypeStruct((N_NODES, DIM), jnp.float32)],
    )(xw1, deg_parts)


def _tc_mid(acc1, tab1, x, dinv_b, b1, gamma1, beta1, W2):
    def body(acc_ref, tab_ref, x_ref, dinv_ref, b_ref, gam_ref, bet_ref,
             w_ref, h_ref, tab2_ref):
        agg = (acc_ref[0, :N_NODES, :] + acc_ref[1, :N_NODES, :]
               + tab_ref[:N_NODES, :])
        g = dinv_ref[...] * agg + b_ref[...]
        mu = jnp.mean(g, axis=0, keepdims=True)
        var = jnp.mean((g - mu) ** 2, axis=0, keepdims=True)
        bn = (g - mu) * lax.rsqrt(var + 1e-5) * gam_ref[...] + bet_ref[...]
        h = x_ref[...] + jnp.maximum(bn, 0.0)
        h_ref[...] = h
        xw2 = jnp.dot(h, w_ref[...],
                      preferred_element_type=jnp.float32,
                      precision=lax.Precision.HIGHEST)
        tab2_ref[:N_NODES, :] = xw2 * dinv_ref[...]
        tab2_ref[N_NODES:, :] = jnp.zeros((TROWS - N_NODES, DIM), jnp.float32)

    return pl.pallas_call(
        body,
        out_shape=[jax.ShapeDtypeStruct((N_NODES, DIM), jnp.float32),
                   jax.ShapeDtypeStruct((TROWS, DIM), jnp.float32)],
    )(acc1, tab1, x, dinv_b, b1, gamma1, beta1, W2)


def _tc_post(acc2, tab2, h1, dinv_b, b2, gamma2, beta2):
    def body(acc_ref, tab_ref, h_ref, dinv_ref, b_ref, gam_ref, bet_ref,
             out_ref):
        agg = (acc_ref[0, :N_NODES, :] + acc_ref[1, :N_NODES, :]
               + tab_ref[:N_NODES, :])
        g = dinv_ref[...] * agg + b_ref[...]
        mu = jnp.mean(g, axis=0, keepdims=True)
        var = jnp.mean((g - mu) ** 2, axis=0, keepdims=True)
        bn = (g - mu) * lax.rsqrt(var + 1e-5) * gam_ref[...] + bet_ref[...]
        h = h_ref[...] + bn
        nrm = jnp.maximum(jnp.sqrt(jnp.sum(h * h, axis=1, keepdims=True)),
                          1e-12)
        out_ref[...] = h / nrm

    return pl.pallas_call(
        body,
        out_shape=jax.ShapeDtypeStruct((N_NODES, DIM), jnp.float32),
    )(acc2, tab2, h1, dinv_b, b2, gamma2, beta2)


def kernel(x, W1, b1, gamma1, beta1, W2, b2, gamma2, beta2, edge_index):
    ei = edge_index.astype(jnp.int32)
    pad = jnp.full((DEG_EDGES - N_EDGES,), N_NODES, jnp.int32)
    src_p = jnp.concatenate([ei[0], pad])
    dst_p = jnp.concatenate([ei[1], pad])

    def _split(e):
        e3 = e.reshape(NUM_SUBCORES, F0_CHUNKS + F1_CHUNKS, CHUNK)
        c0 = e3[:, :F0_CHUNKS]
        c1 = jnp.pad(e3[:, F0_CHUNKS:],
                     ((0, 0), (0, F_MAX - F1_CHUNKS), (0, 0)),
                     constant_values=N_NODES)
        return jnp.stack([c0, c1], axis=1).reshape(
            NUM_WORKERS, F_MAX, CHUNK)

    src_w = _split(src_p)
    dst_w = _split(dst_p)
    dst_d = dst_p.reshape(NUM_WORKERS, DEG_CHUNKS, DCHUNK)

    deg_parts = _sc_degree(dst_d)
    xw1 = _tc_mm1(x, W1)
    tab1, dinv_b = _tc_prep1(xw1, deg_parts)
    acc1 = _sc_scatter(tab1, src_w, dst_w)
    h1, tab2 = _tc_mid(acc1, tab1, x, dinv_b,
                       b1.reshape(1, DIM), gamma1.reshape(1, DIM),
                       beta1.reshape(1, DIM), W2)
    acc2 = _sc_scatter(tab2, src_w, dst_w)
    return _tc_post(acc2, tab2, h1, dinv_b,
                    b2.reshape(1, DIM), gamma2.reshape(1, DIM),
                    beta2.reshape(1, DIM))
